# Initial kernel scaffold; baseline (speedup 1.0000x reference)
import jax
import jax.numpy as jnp
from jax.experimental import pallas as pl


def kernel(indices, embedding_weight):
    raise NotImplementedError("write your pallas kernel here")


# speedup vs baseline: 3.1792x; 3.1792x over previous
import functools

import jax
import jax.numpy as jnp
from jax import lax
from jax.experimental import pallas as pl
from jax.experimental.pallas import tpu as pltpu
from jax.experimental.pallas import tpu_sc as plsc

EMBED = 64
TOTAL = 4096 * 200

_info = plsc.get_sparse_core_info()
NC, NS = _info.num_cores, _info.num_subcores
NW = NC * NS
PER_W = TOTAL // NW
CHUNK = 128
NCHUNK = PER_W // CHUNK

_mesh = plsc.VectorSubcoreMesh(core_axis_name="c", subcore_axis_name="s")


@functools.partial(
    pl.kernel,
    mesh=_mesh,
    out_type=jax.ShapeDtypeStruct((TOTAL, EMBED), jnp.float32),
    compiler_params=pltpu.CompilerParams(use_tc_tiling_on_sc=False),
    scratch_types=[
        pltpu.VMEM((CHUNK,), jnp.int32),
        pltpu.VMEM((CHUNK, EMBED), jnp.float32),
        pltpu.SemaphoreType.DMA,
    ],
)
def _gather_kernel(idx_hbm, table_hbm, out_hbm, idx_v, rows_v, sem):
    wid = lax.axis_index("s") * NC + lax.axis_index("c")
    base = pl.multiple_of(wid * PER_W, CHUNK)

    def body(i, _):
        off = pl.multiple_of(base + i * CHUNK, CHUNK)
        pltpu.sync_copy(idx_hbm.at[pl.ds(off, CHUNK)], idx_v)
        pltpu.async_copy(table_hbm.at[idx_v], rows_v, sem).wait()
        pltpu.sync_copy(rows_v, out_hbm.at[pl.ds(off, CHUNK)])
        return 0

    lax.fori_loop(0, NCHUNK, body, 0)


def kernel(indices, embedding_weight):
    flat = indices.reshape(-1)
    out = _gather_kernel(flat, embedding_weight)
    return out.reshape(indices.shape + (EMBED,))


# speedup vs baseline: 4.1060x; 1.2915x over previous
import functools

import jax
import jax.numpy as jnp
from jax import lax
from jax.experimental import pallas as pl
from jax.experimental.pallas import tpu as pltpu
from jax.experimental.pallas import tpu_sc as plsc

EMBED = 64
TOTAL = 4096 * 200

_info = plsc.get_sparse_core_info()
NC, NS = _info.num_cores, _info.num_subcores
NW = NC * NS
PER_W = TOTAL // NW
CHUNK = 128
NCHUNK = PER_W // CHUNK

_mesh = plsc.VectorSubcoreMesh(core_axis_name="c", subcore_axis_name="s")


@functools.partial(
    pl.kernel,
    mesh=_mesh,
    out_type=jax.ShapeDtypeStruct((TOTAL, EMBED), jnp.float32),
    compiler_params=pltpu.CompilerParams(use_tc_tiling_on_sc=False),
    scratch_types=[
        pltpu.VMEM((NCHUNK, CHUNK), jnp.int32),
        pltpu.VMEM((CHUNK, EMBED), jnp.float32),
        pltpu.VMEM((CHUNK, EMBED), jnp.float32),
        pltpu.SemaphoreType.DMA,
        pltpu.SemaphoreType.DMA,
        pltpu.SemaphoreType.DMA,
        pltpu.SemaphoreType.DMA,
    ],
)
def _gather_kernel(idx_hbm, table_hbm, out_hbm, idx_v, rows0, rows1,
                   g0, g1, s0, s1):
    wid = lax.axis_index("s") * NC + lax.axis_index("c")
    base = wid * PER_W
    rows = (rows0, rows1)
    gsem = (g0, g1)
    ssem = (s0, s1)

    pltpu.sync_copy(idx_hbm.at[wid], idx_v)

    def gstart(i, b):
        pltpu.make_async_copy(table_hbm.at[idx_v.at[i]], rows[b],
                              gsem[b]).start()

    def gwait(i, b):
        pltpu.make_async_copy(table_hbm.at[idx_v.at[i]], rows[b],
                              gsem[b]).wait()

    def sstart(i, b):
        pltpu.make_async_copy(rows[b], out_hbm.at[pl.ds(base + i * CHUNK,
                                                        CHUNK)],
                              ssem[b]).start()

    def swait(i, b):
        pltpu.make_async_copy(rows[b], out_hbm.at[pl.ds(base + i * CHUNK,
                                                        CHUNK)],
                              ssem[b]).wait()

    gstart(0, 0)
    gstart(1, 1)
    gwait(0, 0)
    sstart(0, 0)

    def body(j, _):
        i0 = 2 * j
        swait(i0 - 2, 0)
        gstart(i0, 0)
        gwait(i0 - 1, 1)
        sstart(i0 - 1, 1)
        i1 = i0 + 1
        swait(i1 - 2, 1)
        gstart(i1, 1)
        gwait(i1 - 1, 0)
        sstart(i1 - 1, 0)
        return 0

    lax.fori_loop(1, NCHUNK // 2, body, 0)

    gwait(NCHUNK - 1, 1)
    sstart(NCHUNK - 1, 1)
    swait(NCHUNK - 2, 0)
    swait(NCHUNK - 1, 1)


def kernel(indices, embedding_weight):
    idx = indices.reshape(NW, NCHUNK, CHUNK)
    out = _gather_kernel(idx, embedding_weight)
    return out.reshape(indices.shape + (EMBED,))


# speedup vs baseline: 4.2637x; 1.0384x over previous
import functools

import jax
import jax.numpy as jnp
from jax import lax
from jax.experimental import pallas as pl
from jax.experimental.pallas import tpu as pltpu
from jax.experimental.pallas import tpu_sc as plsc

EMBED = 64
TOTAL = 4096 * 200

_info = plsc.get_sparse_core_info()
NC, NS = _info.num_cores, _info.num_subcores
NW = NC * NS
PER_W = TOTAL // NW
CHUNK = 512
NCHUNK = PER_W // CHUNK

_mesh = plsc.VectorSubcoreMesh(core_axis_name="c", subcore_axis_name="s")


@functools.partial(
    pl.kernel,
    mesh=_mesh,
    out_type=jax.ShapeDtypeStruct((TOTAL, EMBED), jnp.float32),
    compiler_params=pltpu.CompilerParams(use_tc_tiling_on_sc=False),
    scratch_types=[
        pltpu.VMEM((NCHUNK, CHUNK), jnp.int32),
        pltpu.VMEM((CHUNK, EMBED), jnp.float32),
        pltpu.VMEM((CHUNK, EMBED), jnp.float32),
        pltpu.SemaphoreType.DMA,
        pltpu.SemaphoreType.DMA,
        pltpu.SemaphoreType.DMA,
        pltpu.SemaphoreType.DMA,
    ],
)
def _gather_kernel(idx_hbm, table_hbm, out_hbm, idx_v, rows0, rows1,
                   g0, g1, s0, s1):
    wid = lax.axis_index("s") * NC + lax.axis_index("c")
    base = wid * PER_W
    rows = (rows0, rows1)
    gsem = (g0, g1)
    ssem = (s0, s1)

    pltpu.sync_copy(idx_hbm.at[wid], idx_v)

    def gstart(i, b):
        pltpu.make_async_copy(table_hbm.at[idx_v.at[i]], rows[b],
                              gsem[b]).start()

    def gwait(i, b):
        pltpu.make_async_copy(table_hbm.at[idx_v.at[i]], rows[b],
                              gsem[b]).wait()

    def sstart(i, b):
        pltpu.make_async_copy(rows[b], out_hbm.at[pl.ds(base + i * CHUNK,
                                                        CHUNK)],
                              ssem[b]).start()

    def swait(i, b):
        pltpu.make_async_copy(rows[b], out_hbm.at[pl.ds(base + i * CHUNK,
                                                        CHUNK)],
                              ssem[b]).wait()

    gstart(0, 0)
    gstart(1, 1)
    gwait(0, 0)
    sstart(0, 0)

    def body(j, _):
        i0 = 2 * j
        swait(i0 - 2, 0)
        gstart(i0, 0)
        gwait(i0 - 1, 1)
        sstart(i0 - 1, 1)
        i1 = i0 + 1
        swait(i1 - 2, 1)
        gstart(i1, 1)
        gwait(i1 - 1, 0)
        sstart(i1 - 1, 0)
        return 0

    lax.fori_loop(1, NCHUNK // 2, body, 0)

    gwait(NCHUNK - 1, 1)
    sstart(NCHUNK - 1, 1)
    swait(NCHUNK - 2, 0)
    swait(NCHUNK - 1, 1)


def kernel(indices, embedding_weight):
    idx = indices.reshape(NW, NCHUNK, CHUNK)
    out = _gather_kernel(idx, embedding_weight)
    return out.reshape(indices.shape + (EMBED,))
